# Initial kernel scaffold; baseline (speedup 1.0000x reference)
import jax
import jax.numpy as jnp
from jax.experimental import pallas as pl


def kernel(x, router_w, w1, w2, w3):
    raise NotImplementedError("write your pallas kernel here")


# speedup vs baseline: 1.0984x; 1.0984x over previous
import functools

import jax
import jax.numpy as jnp
from jax import lax
from jax.experimental import pallas as pl
from jax.experimental.pallas import tpu as pltpu

D_MODEL = 1024
D_EXPERT = 1024
E = 8
K = 2
BLK = 256
RB = 512
LANES = 128


def _router_body(x_ref, rw_ref, i1_ref, i2_ref, p1_ref, p2_ref):
    x = x_ref[...]
    logits = jnp.dot(x, rw_ref[...], preferred_element_type=jnp.float32)
    lane = lax.broadcasted_iota(jnp.int32, (RB, LANES), 1)
    neg = jnp.float32(-jnp.inf)
    l = jnp.where(lane < E, logits, neg)
    m1 = jnp.max(l, axis=1, keepdims=True)
    i1 = jnp.min(jnp.where(l == m1, lane, LANES), axis=1, keepdims=True)
    l2 = jnp.where(lane == i1, neg, l)
    m2 = jnp.max(l2, axis=1, keepdims=True)
    i2 = jnp.min(jnp.where(l2 == m2, lane, LANES), axis=1, keepdims=True)
    e2 = jnp.exp(m2 - m1)
    denom = 1.0 + e2
    i1_ref[...] = i1
    i2_ref[...] = i2
    p1_ref[...] = 1.0 / denom
    p2_ref[...] = e2 / denom


def _mm_body(be_ref, used_ref, xs_ref, w1_ref, w3_ref, w2_ref, ws_ref, ys_ref):
    g = pl.program_id(0)

    @pl.when(used_ref[g] > 0)
    def _():
        x = xs_ref[...]
        a = jnp.dot(x, w1_ref[0], preferred_element_type=jnp.float32)
        b = jnp.dot(x, w3_ref[0], preferred_element_type=jnp.float32)
        h = a * jax.nn.sigmoid(a) * b
        y = jnp.dot(h, w2_ref[0], preferred_element_type=jnp.float32)
        ys_ref[...] = y * ws_ref[...]


def kernel(x, router_w, w1, w2, w3):
    B, T, C = x.shape
    N = B * T
    NK = N * K
    G = NK // BLK + E
    S = G * BLK
    x_flat = x.reshape(N, C)

    rw_pad = jnp.pad(router_w, ((0, 0), (0, LANES - E)))
    i1, i2, p1, p2 = pl.pallas_call(
        _router_body,
        grid=(N // RB,),
        in_specs=[
            pl.BlockSpec((RB, C), lambda i: (i, 0)),
            pl.BlockSpec((C, LANES), lambda i: (0, 0)),
        ],
        out_specs=[
            pl.BlockSpec((RB, 1), lambda i: (i, 0)),
            pl.BlockSpec((RB, 1), lambda i: (i, 0)),
            pl.BlockSpec((RB, 1), lambda i: (i, 0)),
            pl.BlockSpec((RB, 1), lambda i: (i, 0)),
        ],
        out_shape=[
            jax.ShapeDtypeStruct((N, 1), jnp.int32),
            jax.ShapeDtypeStruct((N, 1), jnp.int32),
            jax.ShapeDtypeStruct((N, 1), jnp.float32),
            jax.ShapeDtypeStruct((N, 1), jnp.float32),
        ],
    )(x_flat, rw_pad)

    a = jnp.stack([i1[:, 0], i2[:, 0]], axis=1).reshape(NK)
    wv = jnp.stack([p1[:, 0], p2[:, 0]], axis=1).reshape(NK)
    oh = (a[:, None] == jnp.arange(E, dtype=jnp.int32)[None, :]).astype(jnp.int32)
    inc = jnp.cumsum(oh, axis=0)
    rank = jnp.take_along_axis(inc, a[:, None], axis=1)[:, 0] - 1
    counts = inc[-1]
    padded = ((counts + BLK - 1) // BLK) * BLK
    ends = jnp.cumsum(padded)
    pad_off = ends - padded
    dest = (pad_off[a] + rank).astype(jnp.int32)
    tok = jnp.arange(NK, dtype=jnp.int32) // K
    gidx = jnp.zeros((S,), jnp.int32).at[dest].set(tok)
    wsort = jnp.zeros((S, 1), jnp.float32).at[dest, 0].set(wv)
    gblk = jnp.arange(G, dtype=jnp.int32) * BLK
    be = jnp.clip(jnp.searchsorted(ends, gblk, side="right"), 0, E - 1).astype(jnp.int32)
    used = (gblk < ends[-1]).astype(jnp.int32)
    sa = dest.reshape(N, K)[:, 0]
    sb = dest.reshape(N, K)[:, 1]

    xs = x_flat[gidx]

    grid_spec = pltpu.PrefetchScalarGridSpec(
        num_scalar_prefetch=2,
        grid=(G,),
        in_specs=[
            pl.BlockSpec((BLK, C), lambda g, be, used: (g, 0)),
            pl.BlockSpec((1, C, D_EXPERT), lambda g, be, used: (be[g], 0, 0)),
            pl.BlockSpec((1, C, D_EXPERT), lambda g, be, used: (be[g], 0, 0)),
            pl.BlockSpec((1, D_EXPERT, C), lambda g, be, used: (be[g], 0, 0)),
            pl.BlockSpec((BLK, 1), lambda g, be, used: (g, 0)),
        ],
        out_specs=pl.BlockSpec((BLK, C), lambda g, be, used: (g, 0)),
    )
    ys = pl.pallas_call(
        _mm_body,
        grid_spec=grid_spec,
        out_shape=jax.ShapeDtypeStruct((S, C), jnp.float32),
    )(be, used, xs, w1, w3, w2, wsort)

    out = ys[sa] + ys[sb]
    return out.reshape(B, T, C)
